# baseline (speedup 1.0000x reference)
import dataclasses
import functools
import math

import jax
import jax.numpy as jnp
from jax import lax
from jax.experimental import pallas as pl
from jax.experimental.pallas import tpu as pltpu
from jax.experimental.pallas import tpu_sc as plsc

D_EMBED = 32
SCALE = math.sqrt(D_EMBED)
LANES = 16
NC, NS = 2, 16
NW = NC * NS
SUB = 128
UNIT_B = 512


def _compiler_params():
  cp = pltpu.CompilerParams(use_tc_tiling_on_sc=False)
  if "needs_layout_passes" in pltpu.CompilerParams.__dataclass_fields__:
    cp = dataclasses.replace(cp, needs_layout_passes=False)
  return cp


@functools.lru_cache(maxsize=None)
def _build(hist: int, batch: int):
  n_bq = batch // UNIT_B
  n_units = hist * n_bq
  k_per_w = n_units // NW
  assert n_units == NW * k_per_w and UNIT_B % SUB == 0

  mesh = plsc.VectorSubcoreMesh(core_axis_name="core", subcore_axis_name="subcore")

  @functools.partial(
      pl.kernel,
      out_type=jax.ShapeDtypeStruct((hist, D_EMBED, batch), jnp.float32),
      mesh=mesh,
      compiler_params=_compiler_params(),
      scratch_types=(
          [pltpu.VMEM((UNIT_B,), jnp.int32)] * 3
          + [pltpu.VMEM((UNIT_B, D_EMBED), jnp.float32)] * 2
          + [pltpu.VMEM((D_EMBED, UNIT_B), jnp.float32)] * 2
          + [pltpu.SemaphoreType.DMA] * 7
      ),
  )
  def emb_kernel(table_hbm, idxt_hbm, out_hbm,
                 i0, i1, i2, a0, a1, b0, b1,
                 s0, s1, s2, s3, s4, s5, s6):
    ibuf, abuf, bbuf = [i0, i1, i2], [a0, a1], [b0, b1]
    isem, gsem, osem = [s0, s1, s2], [s3, s4], [s5, s6]

    wid = lax.axis_index("subcore") * NC + lax.axis_index("core")
    bq = wid % n_bq
    h0 = wid // n_bq
    col0 = bq * UNIT_B
    iota = lax.iota(jnp.int32, LANES)

    def h_of(k):
      return h0 + 4 * k

    def fire_idx(k):
      cp = pltpu.make_async_copy(
          idxt_hbm.at[h_of(k), pl.ds(col0, UNIT_B)], ibuf[k % 3], isem[k % 3])
      cp.start()
      return cp

    def fire_gathers(k):
      cps = []
      for j in range(UNIT_B // SUB):
        cp = pltpu.make_async_copy(
            table_hbm.at[ibuf[k % 3].at[pl.ds(j * SUB, SUB)]],
            abuf[k % 2].at[pl.ds(j * SUB, SUB)],
            gsem[k % 2])
        cp.start()
        cps.append(cp)
      return cps

    def transpose_scale(k):
      a, b = abuf[k % 2], bbuf[k % 2]

      @pl.loop(0, D_EMBED)
      def _(d):
        colv = jnp.full((LANES,), d, dtype=jnp.int32)

        @pl.loop(0, UNIT_B, step=4 * LANES)
        def _(r0):
          for t in range(4):
            rows = iota + (r0 + t * LANES)
            v = plsc.load_gather(a, [rows, colv])
            b.at[d, pl.ds(r0 + t * LANES, LANES)][...] = v * SCALE

    def fire_out(k):
      cp = pltpu.make_async_copy(
          bbuf[k % 2],
          out_hbm.at[h_of(k), pl.ds(0, D_EMBED), pl.ds(col0, UNIT_B)],
          osem[k % 2])
      cp.start()
      return cp

    idx_cp = {0: fire_idx(0), 1: fire_idx(1)}
    idx_cp.pop(0).wait()
    g_cp = {0: fire_gathers(0)}
    o_cp = {}
    for k in range(k_per_w):
      if k + 2 < k_per_w:
        idx_cp[k + 2] = fire_idx(k + 2)
      if k + 1 < k_per_w:
        idx_cp.pop(k + 1).wait()
        g_cp[k + 1] = fire_gathers(k + 1)
      for cp in g_cp.pop(k):
        cp.wait()
      if k >= 2:
        o_cp.pop(k - 2).wait()
      transpose_scale(k)
      o_cp[k] = fire_out(k)
    for k in sorted(o_cp):
      o_cp.pop(k).wait()

  return emb_kernel


def kernel(x, table):
  batch, hist = x.shape
  xt = jnp.transpose(x).astype(jnp.int32)
  out = _build(hist, batch)(table, xt)
  return jnp.transpose(out, (2, 0, 1))


# speedup vs baseline: 1.4960x; 1.4960x over previous
import dataclasses
import functools
import math

import jax
import jax.numpy as jnp
from jax import lax
from jax.experimental import pallas as pl
from jax.experimental.pallas import tpu as pltpu
from jax.experimental.pallas import tpu_sc as plsc

D_EMBED = 32
SCALE = math.sqrt(D_EMBED)
LANES = 16
NC, NS = 2, 16
NW = NC * NS
SUB = 128
UNIT_B = 512


def _compiler_params():
  cp = pltpu.CompilerParams(use_tc_tiling_on_sc=False)
  if "needs_layout_passes" in pltpu.CompilerParams.__dataclass_fields__:
    cp = dataclasses.replace(cp, needs_layout_passes=False)
  return cp


@functools.lru_cache(maxsize=None)
def _build(hist: int, batch: int):
  n_bq = batch // UNIT_B
  n_units = hist * n_bq
  k_per_w = n_units // NW
  assert n_units == NW * k_per_w and UNIT_B % SUB == 0

  mesh = plsc.VectorSubcoreMesh(core_axis_name="core", subcore_axis_name="subcore")

  @functools.partial(
      pl.kernel,
      out_type=jax.ShapeDtypeStruct((hist, D_EMBED, batch), jnp.float32),
      mesh=mesh,
      compiler_params=_compiler_params(),
      scratch_types=(
          [pltpu.VMEM((UNIT_B,), jnp.int32)] * 3
          + [pltpu.VMEM((UNIT_B, D_EMBED), jnp.float32)] * 2
          + [pltpu.VMEM((D_EMBED, UNIT_B + 3), jnp.float32)] * 2
          + [pltpu.SemaphoreType.DMA] * 7
      ),
  )
  def emb_kernel(table_hbm, idxt_hbm, out_hbm,
                 i0, i1, i2, a0, a1, b0, b1,
                 s0, s1, s2, s3, s4, s5, s6):
    ibuf, abuf, bbuf = [i0, i1, i2], [a0, a1], [b0, b1]
    isem, gsem, osem = [s0, s1, s2], [s3, s4], [s5, s6]

    wid = lax.axis_index("subcore") * NC + lax.axis_index("core")
    bq = wid % n_bq
    h0 = wid // n_bq
    col0 = bq * UNIT_B
    iota = lax.iota(jnp.int32, LANES)

    def h_of(k):
      return h0 + 4 * k

    def fire_idx(k):
      cp = pltpu.make_async_copy(
          idxt_hbm.at[h_of(k), pl.ds(col0, UNIT_B)], ibuf[k % 3], isem[k % 3])
      cp.start()
      return cp

    def fire_gathers(k):
      cps = []
      for j in range(UNIT_B // SUB):
        cp = pltpu.make_async_copy(
            table_hbm.at[ibuf[k % 3].at[pl.ds(j * SUB, SUB)]],
            abuf[k % 2].at[pl.ds(j * SUB, SUB)],
            gsem[k % 2])
        cp.start()
        cps.append(cp)
      return cps

    def transpose_scale(k):
      a, b = abuf[k % 2], bbuf[k % 2]

      @pl.loop(0, UNIT_B, step=4)
      def _(r0):
        for t in range(4):
          r = r0 + t
          colv = jnp.full((LANES,), r, dtype=jnp.int32)
          for h2 in range(D_EMBED // LANES):
            v = a.at[r, pl.ds(h2 * LANES, LANES)][...] * SCALE
            plsc.store_scatter(b, [iota + h2 * LANES, colv], v)

    def fire_out(k):
      cp = pltpu.make_async_copy(
          bbuf[k % 2].at[pl.ds(0, D_EMBED), pl.ds(0, UNIT_B)],
          out_hbm.at[h_of(k), pl.ds(0, D_EMBED), pl.ds(col0, UNIT_B)],
          osem[k % 2])
      cp.start()
      return cp

    idx_cp = {0: fire_idx(0), 1: fire_idx(1)}
    idx_cp.pop(0).wait()
    g_cp = {0: fire_gathers(0)}
    o_cp = {}
    for k in range(k_per_w):
      if k + 2 < k_per_w:
        idx_cp[k + 2] = fire_idx(k + 2)
      if k + 1 < k_per_w:
        idx_cp.pop(k + 1).wait()
        g_cp[k + 1] = fire_gathers(k + 1)
      for cp in g_cp.pop(k):
        cp.wait()
      if k >= 2:
        o_cp.pop(k - 2).wait()
      transpose_scale(k)
      o_cp[k] = fire_out(k)
    for k in sorted(o_cp):
      o_cp.pop(k).wait()

  return emb_kernel


def kernel(x, table):
  batch, hist = x.shape
  xt = jnp.transpose(x).astype(jnp.int32)
  out = _build(hist, batch)(table, xt)
  return jnp.transpose(out, (2, 0, 1))
